# Initial kernel scaffold; baseline (speedup 1.0000x reference)
import jax
import jax.numpy as jnp
from jax.experimental import pallas as pl


def kernel(x_source, neighborhood_indices, neighborhood_values):
    raise NotImplementedError("write your pallas kernel here")


# speedup vs baseline: 3.0114x; 3.0114x over previous
import functools

import jax
import jax.numpy as jnp
from jax import lax
from jax.experimental import pallas as pl
from jax.experimental.pallas import tpu as pltpu
from jax.experimental.pallas import tpu_sc as plsc

_N = 10000
_D = 128
_E = 320000
_NC = 2
_NS = 16
_H = _D // _NC
_CHUNK = 128
_CHUNKS = 158
_EPT = _CHUNK * _CHUNKS
_EPAD = _EPT * _NS
_STRIPE = _N // _NS

_mesh = plsc.VectorSubcoreMesh(
    core_axis_name="c", subcore_axis_name="s", num_cores=_NC, num_subcores=_NS
)


@functools.partial(
    pl.kernel,
    out_type=jax.ShapeDtypeStruct((_N, _D), jnp.float32),
    mesh=_mesh,
    compiler_params=pltpu.CompilerParams(use_tc_tiling_on_sc=False),
    scratch_types=[
        pltpu.VMEM((_CHUNKS, _CHUNK), jnp.int32),
        pltpu.VMEM((_CHUNKS, _CHUNK), jnp.int32),
        pltpu.VMEM((_CHUNKS, _CHUNK), jnp.float32),
        pltpu.VMEM((_CHUNK, _H), jnp.float32),
        pltpu.VMEM_SHARED((_N, _H), jnp.float32),
        pltpu.SemaphoreType.DMA,
    ],
)
def _mp_sc(xs_hbm, src_hbm, dst_hbm, val_hbm, out_hbm,
           src_v, dst_v, val_v, buf, acc, gsem):
    c = lax.axis_index("c")
    s = lax.axis_index("s")

    pltpu.sync_copy(src_hbm.at[c, s], src_v)
    pltpu.sync_copy(dst_hbm.at[s], dst_v)
    pltpu.sync_copy(val_hbm.at[s], val_v)

    def _zrow(r, carry):
        for d in range(_H // 16):
            buf[r, pl.ds(d * 16, 16)] = jnp.zeros((16,), jnp.float32)
        return carry
    lax.fori_loop(0, _CHUNK, _zrow, 0)
    row0 = s * _STRIPE
    for i in range(5):
        pltpu.sync_copy(buf.at[pl.ds(0, 125)], acc.at[pl.ds(row0 + i * 125, 125)])
    plsc.subcore_barrier()

    def _scale(ci):
        def g_body(g, carry):
            vals16 = val_v[ci, pl.ds(g * 16, 16)]
            for j in range(16):
                e = g * 16 + j
                vv = lax.gather(
                    vals16,
                    jnp.full((16, 1), j, jnp.int32),
                    lax.GatherDimensionNumbers(
                        offset_dims=(), collapsed_slice_dims=(0,),
                        start_index_map=(0,)),
                    (1,),
                    mode=lax.GatherScatterMode.PROMISE_IN_BOUNDS,
                )
                for d in range(_H // 16):
                    sl = pl.ds(d * 16, 16)
                    buf[e, sl] = buf[e, sl] * vv
            return carry
        lax.fori_loop(0, _CHUNK // 16, g_body, 0)

    def chunk_body(ci, carry):
        pltpu.async_copy(xs_hbm.at[src_v.at[ci]], buf, gsem).wait()
        _scale(ci)
        pltpu.sync_copy(buf, acc.at[dst_v.at[ci]], add=True)
        return carry
    lax.fori_loop(0, _CHUNKS, chunk_body, 0)

    plsc.subcore_barrier()
    pltpu.sync_copy(
        acc.at[pl.ds(row0, _STRIPE)],
        out_hbm.at[pl.ds(row0, _STRIPE), pl.ds(c * _H, _H)],
    )


def kernel(x_source, neighborhood_indices, neighborhood_values):
    dst = neighborhood_indices[0].astype(jnp.int32)
    src = neighborhood_indices[1].astype(jnp.int32)
    val = neighborhood_values.astype(jnp.float32)
    pad = _EPAD - _E
    srcp = jnp.pad(src, (0, pad))
    dstp = jnp.pad(dst, (0, pad))
    valp = jnp.pad(val, (0, pad))
    src2 = jnp.stack([srcp, srcp + _N]).reshape(_NC, _NS, _CHUNKS, _CHUNK)
    dst2 = dstp.reshape(_NS, _CHUNKS, _CHUNK)
    val2 = valp.reshape(_NS, _CHUNKS, _CHUNK)
    xs = jnp.concatenate([x_source[:, :_H], x_source[:, _H:]], axis=0)
    return _mp_sc(xs, src2, dst2, val2)


# speedup vs baseline: 4.2763x; 1.4200x over previous
import functools

import jax
import jax.numpy as jnp
from jax import lax
from jax.experimental import pallas as pl
from jax.experimental.pallas import tpu as pltpu
from jax.experimental.pallas import tpu_sc as plsc

_N = 10000
_D = 128
_E = 320000
_NC = 2
_NS = 16
_H = _D // _NC
_CHUNK = 128
_CHUNKS = 160
_NBUF = 4
_PHASES = 2
_PCH = _CHUNKS // _PHASES
_EPT = _CHUNK * _CHUNKS
_EPAD = _EPT * _NS
_STRIPE = _N // _NS

_mesh = plsc.VectorSubcoreMesh(
    core_axis_name="c", subcore_axis_name="s", num_cores=_NC, num_subcores=_NS
)


@functools.partial(
    pl.kernel,
    out_type=jax.ShapeDtypeStruct((_N, _D), jnp.float32),
    mesh=_mesh,
    compiler_params=pltpu.CompilerParams(use_tc_tiling_on_sc=False),
    scratch_types=[
        pltpu.VMEM((_PCH, _CHUNK), jnp.int32),
        pltpu.VMEM((_PCH, _CHUNK), jnp.int32),
        pltpu.VMEM((_PCH, _CHUNK), jnp.float32),
        [pltpu.VMEM((_CHUNK, _H), jnp.float32) for _ in range(_NBUF)],
        [pltpu.SemaphoreType.DMA for _ in range(_NBUF)],
        [pltpu.SemaphoreType.DMA for _ in range(_NBUF)],
        pltpu.VMEM_SHARED((_N, _H), jnp.float32),
    ],
)
def _mp_sc(xs_hbm, src_hbm, dst_hbm, val_hbm, out_hbm,
           src_v, dst_v, val_v, bufs, gsems, ssems, acc):
    c = lax.axis_index("c")
    s = lax.axis_index("s")

    def _zrow(r, carry):
        for d in range(_H // 16):
            bufs[0][r, pl.ds(d * 16, 16)] = jnp.zeros((16,), jnp.float32)
        return carry
    lax.fori_loop(0, _CHUNK, _zrow, 0)
    row0 = s * _STRIPE
    for i in range(5):
        pltpu.sync_copy(bufs[0].at[pl.ds(0, 125)],
                        acc.at[pl.ds(row0 + i * 125, 125)])
    plsc.subcore_barrier()

    def _scale(buf, ci):
        def g_body(g, carry):
            vals16 = val_v[ci, pl.ds(g * 16, 16)]
            for j in range(16):
                e = g * 16 + j
                vv = lax.gather(
                    vals16,
                    jnp.full((16, 1), j, jnp.int32),
                    lax.GatherDimensionNumbers(
                        offset_dims=(), collapsed_slice_dims=(0,),
                        start_index_map=(0,)),
                    (1,),
                    mode=lax.GatherScatterMode.PROMISE_IN_BOUNDS,
                )
                for d in range(_H // 16):
                    sl = pl.ds(d * 16, 16)
                    buf[e, sl] = buf[e, sl] * vv
            return carry
        lax.fori_loop(0, _CHUNK // 16, g_body, 0)

    def _gather(ci, b):
        pltpu.async_copy(xs_hbm.at[src_v.at[ci]], bufs[b], gsems[b])

    def _scatter(ci, b):
        pltpu.async_copy(bufs[b], acc.at[dst_v.at[ci]], ssems[b], add=True)

    for ph in range(_PHASES):
        pltpu.sync_copy(src_hbm.at[c, s, pl.ds(ph * _PCH, _PCH)], src_v)
        pltpu.sync_copy(dst_hbm.at[s, pl.ds(ph * _PCH, _PCH)], dst_v)
        pltpu.sync_copy(val_hbm.at[s, pl.ds(ph * _PCH, _PCH)], val_v)
        _gather(0, 0)
        _gather(1, 1)

        def ring_body(p, carry):
            i0 = p * _NBUF
            for b in range(_NBUF):
                i = i0 + b
                pltpu.make_async_copy(xs_hbm.at[src_v.at[i]], bufs[b],
                                      gsems[b]).wait()
                _scale(bufs[b], i)
                _scatter(i, b)
                j = i + 2
                bj = (b + 2) % _NBUF

                @pl.when(j < _PCH)
                def _():
                    @pl.when(j >= _NBUF)
                    def _():
                        pltpu.make_async_copy(bufs[bj],
                                              acc.at[dst_v.at[j - _NBUF]],
                                              ssems[bj]).wait()
                    _gather(j, bj)
            return carry
        lax.fori_loop(0, _PCH // _NBUF, ring_body, 0)

        for b in range(_NBUF):
            i = _PCH - _NBUF + b
            pltpu.make_async_copy(bufs[b], acc.at[dst_v.at[i]],
                                  ssems[b]).wait()

    plsc.subcore_barrier()
    pltpu.sync_copy(
        acc.at[pl.ds(row0, _STRIPE)],
        out_hbm.at[pl.ds(row0, _STRIPE), pl.ds(c * _H, _H)],
    )


def kernel(x_source, neighborhood_indices, neighborhood_values):
    dst = neighborhood_indices[0].astype(jnp.int32)
    src = neighborhood_indices[1].astype(jnp.int32)
    val = neighborhood_values.astype(jnp.float32)
    pad = _EPAD - _E
    srcp = jnp.pad(src, (0, pad))
    dstp = jnp.pad(dst, (0, pad))
    valp = jnp.pad(val, (0, pad))
    src2 = jnp.stack([srcp, srcp + _N]).reshape(_NC, _NS, _CHUNKS, _CHUNK)
    dst2 = dstp.reshape(_NS, _CHUNKS, _CHUNK)
    val2 = valp.reshape(_NS, _CHUNKS, _CHUNK)
    xs = jnp.concatenate([x_source[:, :_H], x_source[:, _H:]], axis=0)
    return _mp_sc(xs, src2, dst2, val2)


# speedup vs baseline: 5.9150x; 1.3832x over previous
import functools

import jax
import jax.numpy as jnp
from jax import lax
from jax.experimental import pallas as pl
from jax.experimental.pallas import tpu as pltpu
from jax.experimental.pallas import tpu_sc as plsc

_N = 10000
_D = 128
_E = 320000
_NC = 2
_NS = 16
_H = _D // _NC
_CHUNK = 128
_CHUNKS = 160
_NBUF = 4
_PHASES = 2
_PCH = _CHUNKS // _PHASES
_EPT = _CHUNK * _CHUNKS
_EPAD = _EPT * _NS
_STRIPE = _N // _NS

_mesh = plsc.VectorSubcoreMesh(
    core_axis_name="c", subcore_axis_name="s", num_cores=_NC, num_subcores=_NS
)


@functools.partial(
    pl.kernel,
    out_type=jax.ShapeDtypeStruct((_N, _D), jnp.float32),
    mesh=_mesh,
    compiler_params=pltpu.CompilerParams(use_tc_tiling_on_sc=False),
    scratch_types=[
        pltpu.VMEM((_PCH, _CHUNK), jnp.int32),
        pltpu.VMEM((_PCH, _CHUNK), jnp.int32),
        pltpu.VMEM((_PCH, _CHUNK), jnp.float32),
        [pltpu.VMEM((_CHUNK, _H), jnp.float32) for _ in range(_NBUF)],
        [pltpu.SemaphoreType.DMA for _ in range(_NBUF)],
        [pltpu.SemaphoreType.DMA for _ in range(_NBUF)],
        pltpu.VMEM_SHARED((_N, _H), jnp.float32),
    ],
)
def _mp_sc(xs_hbm, src_hbm, dst_hbm, val_hbm, out_hbm,
           src_v, dst_v, val_v, bufs, gsems, ssems, acc):
    c = lax.axis_index("c")
    s = lax.axis_index("s")

    def _zrow(r, carry):
        for d in range(_H // 16):
            bufs[0][r, pl.ds(d * 16, 16)] = jnp.zeros((16,), jnp.float32)
        return carry
    lax.fori_loop(0, _CHUNK, _zrow, 0)
    row0 = s * _STRIPE
    for i in range(5):
        pltpu.sync_copy(bufs[0].at[pl.ds(0, 125)],
                        acc.at[pl.ds(row0 + i * 125, 125)])
    plsc.subcore_barrier()

    def _scale(buf, ci):
        def g_body(g):
            vals16 = val_v[ci, pl.ds(g * 16, 16)]
            for j in range(16):
                e = g * 16 + j
                vv = jnp.full((16,), vals16[j], jnp.float32)
                for d in range(_H // 16):
                    sl = pl.ds(d * 16, 16)
                    buf[e, sl] = buf[e, sl] * vv
        plsc.parallel_loop(0, _CHUNK // 16, 1, unroll=2)(g_body)

    def _gather(ci, b):
        pltpu.async_copy(xs_hbm.at[src_v.at[ci]], bufs[b], gsems[b])

    def _scatter(ci, b):
        pltpu.async_copy(bufs[b], acc.at[dst_v.at[ci]], ssems[b], add=True)

    for ph in range(_PHASES):
        pltpu.sync_copy(src_hbm.at[c, s, pl.ds(ph * _PCH, _PCH)], src_v)
        pltpu.sync_copy(dst_hbm.at[s, pl.ds(ph * _PCH, _PCH)], dst_v)
        pltpu.sync_copy(val_hbm.at[s, pl.ds(ph * _PCH, _PCH)], val_v)
        _gather(0, 0)
        _gather(1, 1)

        def ring_body(p, carry):
            i0 = p * _NBUF
            for b in range(_NBUF):
                i = i0 + b
                pltpu.make_async_copy(xs_hbm.at[src_v.at[i]], bufs[b],
                                      gsems[b]).wait()
                _scale(bufs[b], i)
                _scatter(i, b)
                j = i + 2
                bj = (b + 2) % _NBUF

                @pl.when(j < _PCH)
                def _():
                    @pl.when(j >= _NBUF)
                    def _():
                        pltpu.make_async_copy(bufs[bj],
                                              acc.at[dst_v.at[j - _NBUF]],
                                              ssems[bj]).wait()
                    _gather(j, bj)
            return carry
        lax.fori_loop(0, _PCH // _NBUF, ring_body, 0)

        for b in range(_NBUF):
            i = _PCH - _NBUF + b
            pltpu.make_async_copy(bufs[b], acc.at[dst_v.at[i]],
                                  ssems[b]).wait()

    plsc.subcore_barrier()
    pltpu.sync_copy(
        acc.at[pl.ds(row0, _STRIPE)],
        out_hbm.at[pl.ds(row0, _STRIPE), pl.ds(c * _H, _H)],
    )


def kernel(x_source, neighborhood_indices, neighborhood_values):
    dst = neighborhood_indices[0].astype(jnp.int32)
    src = neighborhood_indices[1].astype(jnp.int32)
    val = neighborhood_values.astype(jnp.float32)
    pad = _EPAD - _E
    srcp = jnp.pad(src, (0, pad))
    dstp = jnp.pad(dst, (0, pad))
    valp = jnp.pad(val, (0, pad))
    src2 = jnp.stack([srcp, srcp + _N]).reshape(_NC, _NS, _CHUNKS, _CHUNK)
    dst2 = dstp.reshape(_NS, _CHUNKS, _CHUNK)
    val2 = valp.reshape(_NS, _CHUNKS, _CHUNK)
    xs = jnp.concatenate([x_source[:, :_H], x_source[:, _H:]], axis=0)
    return _mp_sc(xs, src2, dst2, val2)
